# baseline (speedup 1.0000x reference)
import jax
import jax.numpy as jnp
from jax.experimental import pallas as pl
from jax.experimental.pallas import tpu as pltpu

_BS, _HALO, _HEADS = 8, 4, 4
_R = _BS + 2 * _HALO
_D = 64
_NH = 8
_NEG = -1e4


def _strip_kernel(s1_ref, s2_ref, wq_ref, wkv_ref, wo_ref, bo_ref,
                  rwmh_ref, msel_ref, tcomb_ref, wc_ref, bc_ref, out_ref):
    C = 256
    i = pl.program_id(0) % _NH
    scale = _D ** -0.5
    bf = jnp.bfloat16

    strip = jnp.concatenate([s1_ref[0], s2_ref[0]], axis=0)
    pix = strip.reshape(16 * 80, C)
    kv = jnp.dot(pix, wkv_ref[...], preferred_element_type=jnp.float32)
    kvw = kv.astype(bf).reshape(16, 80, 2 * C)

    qg = strip[4:12, 4:68, :].reshape(512, C)
    q = jnp.dot(qg, wq_ref[...], preferred_element_type=jnp.float32) * scale
    qb = q.astype(bf)

    relf = []
    for h in range(_HEADS):
        p = jnp.dot(qb[:, h * _D:(h + 1) * _D], rwmh_ref[...],
                    preferred_element_type=jnp.float32)
        pm = p.astype(bf) * msel_ref[...]
        relf.append(jnp.dot(pm, tcomb_ref[...],
                            preferred_element_type=jnp.float32))

    lane = jax.lax.broadcasted_iota(jnp.int32, (1, 256), 1)
    ki, kj = lane // 16, lane % 16
    prow = 8 * i + ki
    rmask = jnp.where((prow >= 4) & (prow < 68), 0.0, _NEG)

    wins, madds = [], []
    for j in range(_NH):
        wins.append(kvw[:, 8 * j:8 * j + 16, :].reshape(256, 2 * C))
        pcol = 8 * j + kj
        madds.append(rmask + jnp.where((pcol >= 4) & (pcol < 68), 0.0, _NEG))

    sims = []
    for j in range(_NH):
        for h in range(_HEADS):
            qblk = qb[:, h * _D:(h + 1) * _D].reshape(8, 8, 8, _D)[:, j].reshape(64, _D)
            sims.append(jax.lax.dot_general(
                qblk, wins[j][:, h * _D:(h + 1) * _D],
                (((1,), (1,)), ((), ())),
                preferred_element_type=jnp.float32))

    ps = []
    for j in range(_NH):
        for h in range(_HEADS):
            rblk = relf[h].reshape(8, 8, 8, 256)[:, j].reshape(64, 256)
            ps.append(jnp.exp(sims[j * _HEADS + h] + rblk + madds[j]))

    ssums = [jnp.sum(p, axis=-1, keepdims=True) for p in ps]
    pbs = [p.astype(bf) for p in ps]

    os_ = []
    for j in range(_NH):
        for h in range(_HEADS):
            vh = wins[j][:, C + h * _D:C + (h + 1) * _D]
            os_.append(jnp.dot(pbs[j * _HEADS + h], vh,
                               preferred_element_type=jnp.float32))

    outs = []
    for j in range(_NH):
        outs.append(jnp.concatenate(
            [os_[j * _HEADS + h] / ssums[j * _HEADS + h]
             for h in range(_HEADS)], axis=1))

    ys = jnp.concatenate(outs, axis=0)
    yo = jnp.dot(ys.astype(bf), wo_ref[...],
                 preferred_element_type=jnp.float32) + bo_ref[...]
    conv = jnp.dot(yo.astype(bf), wc_ref[...],
                   preferred_element_type=jnp.float32) + bc_ref[...]
    for j in range(_NH):
        out_ref[0, 0, :, 8 * j:8 * j + 8, :] = \
            conv[64 * j:64 * (j + 1), :].reshape(_BS, _BS, 4 * C)


def kernel(x, wq, wkv, wo, bo, rel_h, rel_w, w_conv, b_conv):
    B, C, H, W = x.shape
    nh = H // _BS
    bf = jnp.bfloat16

    xt = jnp.transpose(x, (0, 2, 3, 1))
    xp = jnp.pad(xt, ((0, 0), (_HALO, _HALO), (_HALO, 3 * _HALO),
                      (0, 0))).astype(bf)

    ry = jnp.arange(_R)[None, :] - jnp.arange(_BS)[:, None] + (_R - 1)
    rwm = jnp.transpose(rel_w[ry], (2, 0, 1)).reshape(_D, 128)
    rhm = jnp.transpose(rel_h[ry], (2, 0, 1)).reshape(_D, 128)
    rwmh = jnp.concatenate([rwm, rhm], axis=1).astype(bf)

    pixi = jnp.arange(512)
    cc = jnp.arange(128) // 16
    mw = (pixi[:, None] % 8) == cc[None, :]
    mh = (pixi[:, None] // 64) == cc[None, :]
    msel = jnp.concatenate([mw, mh], axis=1).astype(bf)

    lane = jnp.arange(256)
    tw = (lane[None, :] % 16) == (jnp.arange(128) % 16)[:, None]
    th = (lane[None, :] // 16) == (jnp.arange(128) % 16)[:, None]
    tcomb = jnp.concatenate([tw, th], axis=0).astype(bf)

    grid = (B * nh,)
    const = lambda s: (0, 0)
    out6 = pl.pallas_call(
        _strip_kernel,
        grid=grid,
        in_specs=[
            pl.BlockSpec((1, _BS, 80, C), lambda s: (s // _NH, s % _NH, 0, 0)),
            pl.BlockSpec((1, _BS, 80, C), lambda s: (s // _NH, s % _NH + 1, 0, 0)),
            pl.BlockSpec((C, C), const),
            pl.BlockSpec((C, 2 * C), const),
            pl.BlockSpec((C, C), const),
            pl.BlockSpec((1, C), const),
            pl.BlockSpec((_D, 256), const),
            pl.BlockSpec((512, 256), const),
            pl.BlockSpec((256, 256), const),
            pl.BlockSpec((C, 4 * C), const),
            pl.BlockSpec((1, 4 * C), const),
        ],
        out_specs=pl.BlockSpec((1, 1, _BS, 8 * _NH, 4 * C),
                               lambda s: (s // _NH, s % _NH, 0, 0, 0)),
        out_shape=jax.ShapeDtypeStruct((B, nh, _BS, 8 * _NH, 4 * C), jnp.float32),
        compiler_params=pltpu.CompilerParams(
            dimension_semantics=("parallel",),
            vmem_limit_bytes=50 * 1024 * 1024,
        ),
    )(xp, xp, wq.T.astype(bf), wkv.T.astype(bf), wo.T.astype(bf),
      bo.reshape(1, C), rwmh, msel, tcomb,
      w_conv.T.astype(bf), b_conv.reshape(1, 4 * C))

    out = out6.reshape(B, nh, _BS, W, C, 2, 2)
    out = out.transpose(0, 4, 1, 2, 5, 3, 6).reshape(B, C, 2 * H, 2 * W)
    return out


# speedup vs baseline: 1.0314x; 1.0314x over previous
import jax
import jax.numpy as jnp
from jax.experimental import pallas as pl
from jax.experimental.pallas import tpu as pltpu

_BS, _HALO, _HEADS = 8, 4, 4
_R = _BS + 2 * _HALO
_D = 64
_NH = 8
_NEG = -1e4


def _strip_kernel(s1_ref, s2_ref, wq_ref, wkv_ref, wo_ref, bo_ref,
                  rwmh_ref, msel_ref, tcomb_ref, wc_ref, bc_ref, out_ref):
    C = 256
    i = pl.program_id(0) % _NH
    scale = _D ** -0.5
    bf = jnp.bfloat16

    strip = jnp.concatenate([s1_ref[0], s2_ref[0]], axis=0)
    pix = strip.reshape(16 * 80, C)
    kv = jnp.dot(pix, wkv_ref[...], preferred_element_type=jnp.float32)
    kvw = kv.astype(bf).reshape(16, 80, 2 * C)

    qg = strip[4:12, 4:68, :].reshape(512, C)
    q = jnp.dot(qg, wq_ref[...], preferred_element_type=jnp.float32) * scale
    qb = q.astype(bf)

    relf = []
    for h in range(_HEADS):
        p = jnp.dot(qb[:, h * _D:(h + 1) * _D], rwmh_ref[...],
                    preferred_element_type=jnp.float32)
        pm = p.astype(bf) * msel_ref[...]
        relf.append(jnp.dot(pm, tcomb_ref[...],
                            preferred_element_type=jnp.float32))

    lane = jax.lax.broadcasted_iota(jnp.int32, (1, 256), 1)
    ki, kj = lane // 16, lane % 16
    prow = 8 * i + ki
    rmask = jnp.where((prow >= 4) & (prow < 68), 0.0, _NEG)

    wins, madds = [], []
    for j in range(_NH):
        wins.append(kvw[:, 8 * j:8 * j + 16, :].reshape(256, 2 * C))
        pcol = 8 * j + kj
        madds.append(rmask + jnp.where((pcol >= 4) & (pcol < 68), 0.0, _NEG))

    sims = []
    for j in range(_NH):
        for h in range(_HEADS):
            qblk = qb[:, h * _D:(h + 1) * _D].reshape(8, 8, 8, _D)[:, j].reshape(64, _D)
            sims.append(jax.lax.dot_general(
                qblk, wins[j][:, h * _D:(h + 1) * _D],
                (((1,), (1,)), ((), ())),
                preferred_element_type=jnp.float32))

    ps = []
    for j in range(_NH):
        for h in range(_HEADS):
            rblk = relf[h].reshape(8, 8, 8, 256)[:, j].reshape(64, 256)
            ps.append(jnp.exp(sims[j * _HEADS + h] + rblk + madds[j]))

    ssums = [jnp.sum(p, axis=-1, keepdims=True) for p in ps]
    pbs = [p.astype(bf) for p in ps]

    os_ = []
    for j in range(_NH):
        for h in range(_HEADS):
            vh = wins[j][:, C + h * _D:C + (h + 1) * _D]
            os_.append(jnp.dot(pbs[j * _HEADS + h], vh,
                               preferred_element_type=jnp.float32))

    outs = []
    for j in range(_NH):
        outs.append(jnp.concatenate(
            [os_[j * _HEADS + h] / ssums[j * _HEADS + h]
             for h in range(_HEADS)], axis=1))

    ys = jnp.concatenate(outs, axis=0)
    yo = jnp.dot(ys.astype(bf), wo_ref[...],
                 preferred_element_type=jnp.float32) + bo_ref[...]
    conv = jnp.dot(yo.astype(bf), wc_ref[...],
                   preferred_element_type=jnp.float32) + bc_ref[...]
    convb = conv.astype(bf)
    for j in range(_NH):
        out_ref[0, 0, :, 8 * j:8 * j + 8, :] = \
            convb[64 * j:64 * (j + 1), :].reshape(_BS, _BS, 4 * C)


def kernel(x, wq, wkv, wo, bo, rel_h, rel_w, w_conv, b_conv):
    B, C, H, W = x.shape
    nh = H // _BS
    bf = jnp.bfloat16

    xt = jnp.transpose(x, (0, 2, 3, 1))
    xp = jnp.pad(xt, ((0, 0), (_HALO, _HALO), (_HALO, 3 * _HALO),
                      (0, 0))).astype(bf)

    ry = jnp.arange(_R)[None, :] - jnp.arange(_BS)[:, None] + (_R - 1)
    rwm = jnp.transpose(rel_w[ry], (2, 0, 1)).reshape(_D, 128)
    rhm = jnp.transpose(rel_h[ry], (2, 0, 1)).reshape(_D, 128)
    rwmh = jnp.concatenate([rwm, rhm], axis=1).astype(bf)

    pixi = jnp.arange(512)
    cc = jnp.arange(128) // 16
    mw = (pixi[:, None] % 8) == cc[None, :]
    mh = (pixi[:, None] // 64) == cc[None, :]
    msel = jnp.concatenate([mw, mh], axis=1).astype(bf)

    lane = jnp.arange(256)
    tw = (lane[None, :] % 16) == (jnp.arange(128) % 16)[:, None]
    th = (lane[None, :] // 16) == (jnp.arange(128) % 16)[:, None]
    tcomb = jnp.concatenate([tw, th], axis=0).astype(bf)

    grid = (B * nh,)
    const = lambda s: (0, 0)
    out6 = pl.pallas_call(
        _strip_kernel,
        grid=grid,
        in_specs=[
            pl.BlockSpec((1, _BS, 80, C), lambda s: (s // _NH, s % _NH, 0, 0)),
            pl.BlockSpec((1, _BS, 80, C), lambda s: (s // _NH, s % _NH + 1, 0, 0)),
            pl.BlockSpec((C, C), const),
            pl.BlockSpec((C, 2 * C), const),
            pl.BlockSpec((C, C), const),
            pl.BlockSpec((1, C), const),
            pl.BlockSpec((_D, 256), const),
            pl.BlockSpec((512, 256), const),
            pl.BlockSpec((256, 256), const),
            pl.BlockSpec((C, 4 * C), const),
            pl.BlockSpec((1, 4 * C), const),
        ],
        out_specs=pl.BlockSpec((1, 1, _BS, 8 * _NH, 4 * C),
                               lambda s: (s // _NH, s % _NH, 0, 0, 0)),
        out_shape=jax.ShapeDtypeStruct((B, nh, _BS, 8 * _NH, 4 * C), jnp.bfloat16),
        compiler_params=pltpu.CompilerParams(
            dimension_semantics=("parallel",),
            vmem_limit_bytes=50 * 1024 * 1024,
        ),
    )(xp, xp, wq.T.astype(bf), wkv.T.astype(bf), wo.T.astype(bf),
      bo.reshape(1, C), rwmh, msel, tcomb,
      w_conv.T.astype(bf), b_conv.reshape(1, 4 * C))

    out = out6.reshape(B, nh, _BS, W, C, 2, 2)
    out = out.transpose(0, 4, 1, 2, 5, 3, 6).reshape(B, C, 2 * H, 2 * W)
    return out.astype(jnp.float32)
